# baseline (speedup 1.0000x reference)
import functools

import jax
import jax.numpy as jnp
from jax import lax
from jax.experimental import pallas as pl
from jax.experimental.pallas import tpu as pltpu
from jax.experimental.pallas import tpu_sc as plsc

_VOCAB = 1000000
_CD = 16
_ED = 64
_NB = 16384
_NF = 26
_NFP = 32
_NP = _NB * _NFP

_NC = 2
_NS = 16
_NW = _NC * _NS
_PER_W = _NP // _NW
_PACK = 8
_GR = _PER_W // _PACK

_MM_ROWS = _NP // _PACK
_BB = 512


def _sc_gather(table, idx):
    mesh = plsc.VectorSubcoreMesh(core_axis_name="c", subcore_axis_name="s")

    @functools.partial(
        pl.kernel,
        mesh=mesh,
        out_type=jax.ShapeDtypeStruct((_MM_ROWS, _PACK * _CD), jnp.float32),
        compiler_params=pltpu.CompilerParams(use_tc_tiling_on_sc=False),
        scratch_types=[
            pltpu.VMEM((_PER_W,), jnp.int32),
            pltpu.VMEM((_GR, _CD), jnp.float32),
            pltpu.VMEM((_GR, _CD), jnp.float32),
            pltpu.SemaphoreType.DMA,
            pltpu.SemaphoreType.DMA,
            pltpu.SemaphoreType.DMA,
            pltpu.SemaphoreType.DMA,
        ],
    )
    def k(table_hbm, idx_hbm, out_hbm, idx_v, buf0, buf1, g0, g1, w0, w1):
        wid = lax.axis_index("s") * _NC + lax.axis_index("c")
        pltpu.sync_copy(idx_hbm.at[pl.ds(wid * _PER_W, _PER_W)], idx_v)
        row0 = wid * _GR
        bufs, gsems, wsems = (buf0, buf1), (g0, g1), (w0, w1)
        gd = [None, None]
        wd = [None, None]
        for p in range(_PACK):
            b = p & 1
            if wd[b] is not None:
                wd[b].wait()
            gd[b] = pltpu.async_copy(
                table_hbm.at[idx_v.at[pl.ds(p * _GR, _GR)]], bufs[b], gsems[b]
            )
            if p > 0:
                gd[1 - b].wait()
                wd[1 - b] = pltpu.async_copy(
                    bufs[1 - b],
                    out_hbm.at[
                        pl.ds(row0, _GR), pl.ds((p - 1) * _CD, _CD)
                    ],
                    wsems[1 - b],
                )
        last = (_PACK - 1) & 1
        gd[last].wait()
        wd[last] = pltpu.async_copy(
            bufs[last],
            out_hbm.at[pl.ds(row0, _GR), pl.ds((_PACK - 1) * _CD, _CD)],
            wsems[last],
        )
        wd[0].wait()
        wd[1].wait()

    return k(table, idx)


def _mm_body(e_ref, w_ref, b_ref, o_ref):
    o_ref[...] = (
        jnp.dot(e_ref[...], w_ref[...], preferred_element_type=jnp.float32)
        + b_ref[...]
    )


def _tc_project(emb_p, big_w, bias_p):
    blk = 2048
    return pl.pallas_call(
        _mm_body,
        grid=(_MM_ROWS // blk,),
        in_specs=[
            pl.BlockSpec((blk, _PACK * _CD), lambda i: (i, 0)),
            pl.BlockSpec((_PACK * _CD, _PACK * _ED), lambda i: (0, 0)),
            pl.BlockSpec((1, _PACK * _ED), lambda i: (0, 0)),
        ],
        out_specs=pl.BlockSpec((blk, _PACK * _ED), lambda i: (i, 0)),
        out_shape=jax.ShapeDtypeStruct((_MM_ROWS, _PACK * _ED), jnp.float32),
    )(emb_p, big_w, bias_p)


def kernel(x, table, W, b):
    xi = x.astype(jnp.int32)
    x32 = jnp.concatenate([xi, xi[:, : _NFP - _NF]], axis=1)
    idx = (
        x32.reshape(_NW // 4, _NB // (_NW // 4), _NFP // _PACK, _PACK)
        .transpose(2, 0, 3, 1)
        .reshape(-1)
    )
    emb_p = _sc_gather(table, idx)
    big_w = jnp.kron(jnp.eye(_PACK, dtype=W.dtype), W.T)
    bias_p = jnp.tile(b, _PACK)[None, :]
    out_p = _tc_project(emb_p, big_w, bias_p)
    out4 = out_p.reshape(_NFP // _PACK, _NB, _PACK, _ED)
    return out4.transpose(1, 0, 2, 3).reshape(_NB, _NFP, _ED)[:, :_NF, :]


# speedup vs baseline: 1.1144x; 1.1144x over previous
import functools

import jax
import jax.numpy as jnp
from jax import lax
from jax.experimental import pallas as pl
from jax.experimental.pallas import tpu as pltpu
from jax.experimental.pallas import tpu_sc as plsc

_VOCAB = 1000000
_CD = 16
_ED = 64
_NB = 16384
_NF = 26
_NFP = 32
_NP = _NB * _NFP

_NC = 2
_NS = 16
_NW = _NC * _NS
_PER_W = _NP // _NW
_PACK = 8
_GR = _PER_W // _PACK

_MM_ROWS = _NP // _PACK
_BB = 512


def _sc_gather(table, idx):
    mesh = plsc.VectorSubcoreMesh(core_axis_name="c", subcore_axis_name="s")

    @functools.partial(
        pl.kernel,
        mesh=mesh,
        out_type=jax.ShapeDtypeStruct((_MM_ROWS, _PACK * _CD), jnp.float32),
        compiler_params=pltpu.CompilerParams(use_tc_tiling_on_sc=False),
        scratch_types=[
            pltpu.VMEM((_PER_W,), jnp.int32),
            pltpu.VMEM((_GR, _CD), jnp.float32),
            pltpu.VMEM((_GR, _CD), jnp.float32),
            pltpu.SemaphoreType.DMA,
            pltpu.SemaphoreType.DMA,
            pltpu.SemaphoreType.DMA,
            pltpu.SemaphoreType.DMA,
        ],
    )
    def k(table_hbm, idx_hbm, out_hbm, idx_v, buf0, buf1, g0, g1, w0, w1):
        wid = lax.axis_index("s") * _NC + lax.axis_index("c")
        pltpu.sync_copy(idx_hbm.at[pl.ds(wid * _PER_W, _PER_W)], idx_v)
        row0 = wid * _GR
        bufs, gsems, wsems = (buf0, buf1), (g0, g1), (w0, w1)
        gd = [None, None]
        wd = [None, None]
        for p in range(_PACK):
            b = p & 1
            if wd[b] is not None:
                wd[b].wait()
            gd[b] = pltpu.async_copy(
                table_hbm.at[idx_v.at[pl.ds(p * _GR, _GR)]], bufs[b], gsems[b]
            )
            if p > 0:
                gd[1 - b].wait()
                wd[1 - b] = pltpu.async_copy(
                    bufs[1 - b],
                    out_hbm.at[
                        pl.ds(row0, _GR), pl.ds((p - 1) * _CD, _CD)
                    ],
                    wsems[1 - b],
                )
        last = (_PACK - 1) & 1
        gd[last].wait()
        wd[last] = pltpu.async_copy(
            bufs[last],
            out_hbm.at[pl.ds(row0, _GR), pl.ds((_PACK - 1) * _CD, _CD)],
            wsems[last],
        )
        wd[0].wait()
        wd[1].wait()

    return k(table, idx)


def _perm_body(x_ref, o_ref):
    xt = jnp.concatenate(
        [x_ref[...], x_ref[:, : _NFP - _NF]], axis=1
    ).T
    o_ref[...] = xt.reshape(_NFP // _PACK, 1, _PACK, _NB // _PACK)


def _permute_idx(x):
    nb8 = _NB // _PACK
    out = pl.pallas_call(
        _perm_body,
        grid=(_PACK,),
        in_specs=[pl.BlockSpec((nb8, _NF), lambda i: (i, 0))],
        out_specs=pl.BlockSpec(
            (_NFP // _PACK, 1, _PACK, nb8), lambda i: (0, i, 0, 0)
        ),
        out_shape=jax.ShapeDtypeStruct(
            (_NFP // _PACK, _PACK, _PACK, nb8), jnp.int32
        ),
    )(x)
    return out.reshape(-1)


def _mm_body(e0, e1, e2, e3, w_ref, b_ref, o_ref):
    es = (e0, e1, e2, e3)
    for f in range(_NF):
        q, j = divmod(f, _PACK)
        o_ref[:, f, :] = (
            jnp.dot(es[q][...], w_ref[j], preferred_element_type=jnp.float32)
            + b_ref[...]
        )


def _tc_project(emb_p, w3, b_col):
    nblk = _NB // _BB
    e_specs = [
        pl.BlockSpec(
            (_BB, _PACK * _CD), functools.partial(lambda q, i: (q * nblk + i, 0), q)
        )
        for q in range(_NFP // _PACK)
    ]
    return pl.pallas_call(
        _mm_body,
        grid=(nblk,),
        in_specs=e_specs
        + [
            pl.BlockSpec((_PACK, _PACK * _CD, _ED), lambda i: (0, 0, 0)),
            pl.BlockSpec((1, _ED), lambda i: (0, 0)),
        ],
        out_specs=pl.BlockSpec((_BB, _NF, _ED), lambda i: (i, 0, 0)),
        out_shape=jax.ShapeDtypeStruct((_NB, _NF, _ED), jnp.float32),
    )(emb_p, emb_p, emb_p, emb_p, w3, b_col)


def kernel(x, table, W, b):
    xi = x.astype(jnp.int32)
    idx = _permute_idx(xi)
    emb_p = _sc_gather(table, idx)
    big_w = jnp.kron(jnp.eye(_PACK, dtype=W.dtype), W.T)
    w3 = big_w.reshape(_PACK * _CD, _PACK, _ED).transpose(1, 0, 2)
    b_col = b[None, :]
    return _tc_project(emb_p, w3, b_col)


# speedup vs baseline: 1.1611x; 1.0419x over previous
import functools

import jax
import jax.numpy as jnp
from jax import lax
from jax.experimental import pallas as pl
from jax.experimental.pallas import tpu as pltpu
from jax.experimental.pallas import tpu_sc as plsc

_VOCAB = 1000000
_CD = 16
_ED = 64
_NB = 16384
_NF = 26
_N = _NB * _NF

_NC = 2
_NS = 16
_NW = _NC * _NS
_PER_W = _N // _NW
_CHUNK = 1664
_NCHUNK = _PER_W // _CHUNK

_PACK = 8
_MM_ROWS = _N // _PACK
_MM_BLK = 2048


def _sc_gather(table, idx):
    mesh = plsc.VectorSubcoreMesh(core_axis_name="c", subcore_axis_name="s")

    @functools.partial(
        pl.kernel,
        mesh=mesh,
        out_type=jax.ShapeDtypeStruct((_N, _CD), jnp.float32),
        compiler_params=pltpu.CompilerParams(use_tc_tiling_on_sc=False),
        scratch_types=[
            pltpu.VMEM((_CHUNK,), jnp.int32),
            pltpu.VMEM((_CHUNK, _CD), jnp.float32),
            pltpu.SemaphoreType.DMA,
        ],
    )
    def k(table_hbm, idx_hbm, out_hbm, idx_v, rows_v, sem):
        wid = lax.axis_index("s") * _NC + lax.axis_index("c")
        base = wid * _PER_W
        for c in range(_NCHUNK):
            off = base + c * _CHUNK
            pltpu.sync_copy(idx_hbm.at[pl.ds(off, _CHUNK)], idx_v)
            pltpu.async_copy(table_hbm.at[idx_v], rows_v, sem).wait()
            pltpu.sync_copy(rows_v, out_hbm.at[pl.ds(off, _CHUNK)])

    return k(table, idx)


def _mm_body(e_ref, w_ref, b_ref, o_ref):
    o_ref[...] = (
        jnp.dot(e_ref[...], w_ref[...], preferred_element_type=jnp.float32)
        + b_ref[...]
    )


def _tc_project(emb_p, big_w, bias_p):
    return pl.pallas_call(
        _mm_body,
        grid=(_MM_ROWS // _MM_BLK,),
        in_specs=[
            pl.BlockSpec((_MM_BLK, _PACK * _CD), lambda i: (i, 0)),
            pl.BlockSpec((_PACK * _CD, _PACK * _ED), lambda i: (0, 0)),
            pl.BlockSpec((1, _PACK * _ED), lambda i: (0, 0)),
        ],
        out_specs=pl.BlockSpec((_MM_BLK, _PACK * _ED), lambda i: (i, 0)),
        out_shape=jax.ShapeDtypeStruct((_MM_ROWS, _PACK * _ED), jnp.float32),
    )(emb_p, big_w, bias_p)


def kernel(x, table, W, b):
    idx = x.reshape(-1).astype(jnp.int32)
    emb = _sc_gather(table, idx)
    emb_p = emb.reshape(_MM_ROWS, _PACK * _CD)
    big_w = jnp.kron(jnp.eye(_PACK, dtype=W.dtype), W.T)
    bias_p = jnp.tile(b, _PACK)[None, :]
    out_p = _tc_project(emb_p, big_w, bias_p)
    return out_p.reshape(_NB, _NF, _ED)
